# Initial kernel scaffold; baseline (speedup 1.0000x reference)
import jax
import jax.numpy as jnp
from jax.experimental import pallas as pl


def kernel(x, edge_index, edge_weight, W):
    raise NotImplementedError("write your pallas kernel here")


# speedup vs baseline: 3.3559x; 3.3559x over previous
import jax
import jax.numpy as jnp
from jax import lax
from jax.experimental import pallas as pl
from jax.experimental.pallas import tpu as pltpu
from jax.experimental.pallas import tpu_sc as plsc

_NUM_CORES = 2
_NUM_SUBCORES = 16
_NUM_WORKERS = _NUM_CORES * _NUM_SUBCORES
_LANES = 16
_CHUNK = 128


def _sc_aggregate(x, src, dst, w, edges_per_worker, n_rows):
    feat = x.shape[1]
    rows_per_tile = n_rows // _NUM_SUBCORES
    n_chunks = edges_per_worker // _CHUNK
    jblocks = feat // _LANES

    mesh = plsc.VectorSubcoreMesh(
        core_axis_name="c", subcore_axis_name="s",
        num_cores=_NUM_CORES, num_subcores=_NUM_SUBCORES)

    def body(x_hbm, src_hbm, dst_hbm, w_hbm, out_hbm,
             acc, src_v, dst_v, w_v, rows_v, gsem):
        cid = lax.axis_index("c")
        sid = lax.axis_index("s")
        wid = sid * _NUM_CORES + cid

        zvec = jnp.zeros((_LANES,), jnp.float32)

        def zfill(i, _):
            for j in range(jblocks):
                rows_v[i, pl.ds(j * _LANES, _LANES)] = zvec
            return 0

        lax.fori_loop(0, _CHUNK, zfill, 0)
        row0 = sid * rows_per_tile
        full, rem = divmod(rows_per_tile, _CHUNK)
        for k in range(full):
            pltpu.sync_copy(rows_v, acc.at[pl.ds(row0 + k * _CHUNK, _CHUNK)])
        if rem:
            pltpu.sync_copy(rows_v.at[pl.ds(0, rem)],
                            acc.at[pl.ds(row0 + full * _CHUNK, rem)])
        plsc.subcore_barrier()

        ebase = wid * edges_per_worker

        def chunk(ci, _):
            base = ebase + ci * _CHUNK
            pltpu.sync_copy(src_hbm.at[pl.ds(base, _CHUNK)], src_v)
            pltpu.sync_copy(dst_hbm.at[pl.ds(base, _CHUNK)], dst_v)
            pltpu.sync_copy(w_hbm.at[pl.ds(base, _CHUNK)], w_v)
            pltpu.async_copy(x_hbm.at[src_v], rows_v, gsem).wait()

            def scale(g, _):
                w16 = w_v[pl.ds(g * _LANES, _LANES)]
                for k in range(_LANES):
                    we = w16[k]
                    e = g * _LANES + k
                    for j in range(jblocks):
                        sl = pl.ds(j * _LANES, _LANES)
                        rows_v[e, sl] = rows_v[e, sl] * we
                return 0

            lax.fori_loop(0, _CHUNK // _LANES, scale, 0)
            pltpu.sync_copy(rows_v, acc.at[dst_v], add=True)
            return 0

        lax.fori_loop(0, n_chunks, chunk, 0)
        plsc.subcore_barrier()

        pltpu.sync_copy(acc.at[pl.ds(row0, rows_per_tile)],
                        out_hbm.at[cid, pl.ds(row0, rows_per_tile)])

    fn = pl.kernel(
        body,
        out_type=jax.ShapeDtypeStruct((_NUM_CORES, n_rows, feat),
                                      jnp.float32),
        mesh=mesh,
        scratch_types=[
            pltpu.VMEM_SHARED((n_rows, feat), jnp.float32),
            pltpu.VMEM((_CHUNK,), jnp.int32),
            pltpu.VMEM((_CHUNK,), jnp.int32),
            pltpu.VMEM((_CHUNK,), jnp.float32),
            pltpu.VMEM((_CHUNK, feat), jnp.float32),
            pltpu.SemaphoreType.DMA,
        ],
    )
    return fn(x, src, dst, w)


def _tc_matmul_relu(acc, W):
    n_rows, feat = acc.shape[1], acc.shape[2]
    out_f = W.shape[1]
    block = 1024
    grid = n_rows // block

    def body(a_ref, w_ref, o_ref):
        s = a_ref[0] + a_ref[1]
        o_ref[...] = jnp.maximum(
            jnp.dot(s, w_ref[...], preferred_element_type=jnp.float32), 0.0)

    return pl.pallas_call(
        body,
        grid=(grid,),
        in_specs=[
            pl.BlockSpec((_NUM_CORES, block, feat), lambda i: (0, i, 0)),
            pl.BlockSpec((feat, out_f), lambda i: (0, 0)),
        ],
        out_specs=pl.BlockSpec((block, out_f), lambda i: (i, 0)),
        out_shape=jax.ShapeDtypeStruct((n_rows, out_f), jnp.float32),
    )(acc, W)


def kernel(x, edge_index, edge_weight, W):
    n_nodes = x.shape[0]
    n_edges = edge_index.shape[1]
    grain = _NUM_WORKERS * _CHUNK
    e_pad = -(-n_edges // grain) * grain
    pad = e_pad - n_edges
    src = jnp.concatenate([edge_index[0], jnp.zeros((pad,), jnp.int32)])
    dst = jnp.concatenate([edge_index[1], jnp.zeros((pad,), jnp.int32)])
    w = jnp.concatenate([edge_weight, jnp.zeros((pad,), jnp.float32)])
    rgrain = _NUM_SUBCORES * _CHUNK
    n_rows = -(-n_nodes // rgrain) * rgrain
    acc = _sc_aggregate(x, src, dst, w, e_pad // _NUM_WORKERS, n_rows)
    out = _tc_matmul_relu(acc, W)
    return out[:n_nodes]
